# Initial kernel scaffold; baseline (speedup 1.0000x reference)
import jax
import jax.numpy as jnp
from jax.experimental import pallas as pl


def kernel(s0, s1, edge_index, drop_prob, W_pre, bn_gamma, bn_beta, W_sage, W_gcn):
    raise NotImplementedError("write your pallas kernel here")


# speedup vs baseline: 3.2035x; 3.2035x over previous
import jax
import jax.numpy as jnp
from jax import lax
from jax.experimental import pallas as pl
from jax.experimental.pallas import tpu as pltpu
from jax.experimental.pallas import tpu_sc as plsc

N = 10000
C = 128
E = 320000
F32 = jnp.float32

NSC = 2
NT = 16
NW = NSC * NT
CHUNK = 128
GRP = 8
K1 = 160
K2 = 80
N_PAD = 10112
ZSTRIPE = N_PAD // NT
OSTRIPE_LAST = N - (NT - 1) * ZSTRIPE
DN = 80

BR = 1000
NB = N // BR


def _make_seg_kernel(k_chunks, with_deg):
    mesh = plsc.VectorSubcoreMesh(core_axis_name="c", subcore_axis_name="s")
    out_type = [jax.ShapeDtypeStruct((NSC * N, C), F32)]
    scratch = [
        pltpu.VMEM((GRP, CHUNK), jnp.int32),
        pltpu.VMEM((GRP, CHUNK), jnp.int32),
        pltpu.VMEM((CHUNK, C), F32),
        pltpu.VMEM_SHARED((N_PAD, C), F32),
        pltpu.SemaphoreType.DMA,
    ]
    if with_deg:
        out_type.append(jax.ShapeDtypeStruct((DN, C), F32))
        scratch += [
            pltpu.VMEM((DN, C), F32),
            pltpu.VMEM((DN,), jnp.int32),
            pltpu.VMEM_SHARED((DN, C), F32),
        ]

    def body(*refs):
        if with_deg:
            (table, srcs, dsts, zc, out, deg_out,
             src_v, dst_v, rows_v, acc_sh, sem, deg_v, iden_v, deg_sh) = refs
        else:
            (table, srcs, dsts, zc, out,
             src_v, dst_v, rows_v, acc_sh, sem) = refs
        c = lax.axis_index("c")
        s = lax.axis_index("s")
        w = c * NT + s
        zoff = pl.multiple_of(s * ZSTRIPE, 8)
        pltpu.sync_copy(zc.at[pl.ds(zoff, ZSTRIPE)],
                        acc_sh.at[pl.ds(zoff, ZSTRIPE)])
        if with_deg:
            @pl.when(s == 0)
            def _():
                pltpu.sync_copy(zc.at[pl.ds(0, DN)], deg_sh.at[...])
            zv = jnp.zeros((16,), F32)

            def zrow(i, carry):
                for k in range(C // 16):
                    deg_v[i, pl.ds(k * 16, 16)] = zv
                return carry

            lax.fori_loop(0, DN, zrow, 0)
            for k in range(DN // 16):
                iden_v[pl.ds(k * 16, 16)] = (
                    lax.iota(jnp.int32, 16) + (k * 16))
        plsc.subcore_barrier()

        ones16 = jnp.full((16,), 1.0, F32)

        def group(g, carry):
            goff = pl.multiple_of(g * GRP, GRP)
            pltpu.sync_copy(srcs.at[w, pl.ds(goff, GRP)], src_v)
            pltpu.sync_copy(dsts.at[w, pl.ds(goff, GRP)], dst_v)
            for q in range(GRP):
                pltpu.async_copy(table.at[src_v.at[q]], rows_v, sem).wait()
                pltpu.sync_copy(rows_v, acc_sh.at[dst_v.at[q]], add=True)
                if with_deg:
                    for i in range(CHUNK // 16):
                        d16 = dst_v[q, pl.ds(i * 16, 16)]
                        plsc.addupdate_scatter(
                            deg_v,
                            [lax.shift_right_logical(d16, 7),
                             lax.bitwise_and(d16, 127)],
                            ones16)
            return carry

        lax.fori_loop(0, k_chunks // GRP, group, 0)
        if with_deg:
            pltpu.sync_copy(deg_v, deg_sh.at[iden_v], add=True)
        plsc.subcore_barrier()
        ooff = pl.multiple_of(c * N + s * ZSTRIPE, 8)

        @pl.when(s < NT - 1)
        def _():
            pltpu.sync_copy(acc_sh.at[pl.ds(zoff, ZSTRIPE)],
                            out.at[pl.ds(ooff, ZSTRIPE)])

        @pl.when(s == NT - 1)
        def _():
            pltpu.sync_copy(acc_sh.at[pl.ds((NT - 1) * ZSTRIPE, OSTRIPE_LAST)],
                            out.at[pl.ds(ooff, OSTRIPE_LAST)])

        if with_deg:
            @pl.when((c == 0) & (s == 0))
            def _():
                pltpu.sync_copy(deg_sh, deg_out)

    return pl.kernel(body, out_type=tuple(out_type), mesh=mesh,
                     scratch_types=scratch,
                     compiler_params=pltpu.CompilerParams(
                         needs_layout_passes=False))


def _dot(a, b):
    return jnp.dot(a, b, preferred_element_type=F32)


def _relu(x):
    return jnp.maximum(x, 0.0)


def _pre_kernel(s_ref, w_ref, h_ref, st_ref):
    j = pl.program_id(1)
    h = _dot(s_ref[0], w_ref[0])
    h_ref[0] = h
    colsum = jnp.sum(h, axis=0, keepdims=True)
    colsq = jnp.sum(h * h, axis=0, keepdims=True)
    stats = jnp.concatenate(
        [colsum, colsq, jnp.zeros((6, C), F32)], axis=0)

    @pl.when(j == 0)
    def _():
        st_ref[0] = stats

    @pl.when(j > 0)
    def _():
        st_ref[0] = st_ref[0] + stats


def _norm_kernel(h_ref, st_ref, g_ref, b_ref, p_ref):
    st = st_ref[0]
    mean = st[0:1] * (1.0 / N)
    var = st[1:2] * (1.0 / N) - mean * mean
    scale = g_ref[0, 0:1] * lax.rsqrt(var + 1e-5)
    shift = b_ref[0, 0:1] - mean * scale
    p_ref[0] = _relu(h_ref[0] * scale + shift)


def _mid_kernel(p0_ref, p1_ref, a_ref, deg_ref, ws_ref, wg_ref,
                st2_ref, st3_ref, h6_ref):
    p0 = p0_ref[...]
    p1 = p1_ref[...]
    r = 1.0 / (deg_ref[...] + 1.0)
    m0 = (a_ref[0] + p0) * r
    m1 = (a_ref[1] + p1) * r
    st2_ref[...] = (_relu(_dot(p0, ws_ref[0, 0]) + _dot(m0, ws_ref[0, 1]))
                    + _relu(_dot(m1, wg_ref[0])))
    st3_ref[...] = _relu(_dot(p1, ws_ref[1, 0]) + _dot(m1, ws_ref[1, 1])) + p0
    h6_ref[...] = _relu(_dot(p1, ws_ref[2, 0]) + _dot(m1, ws_ref[2, 1]))


def _fin_kernel(st2_ref, st3_ref, h6_ref, a2_ref, deg_ref, wg_ref, o_ref):
    st2 = st2_ref[...]
    st3 = st3_ref[...]
    r = 1.0 / (deg_ref[...] + 1.0)
    m2 = (a2_ref[0] + a2_ref[1] + st2) * r
    o_ref[:, 0:C] = st2
    o_ref[:, C:2 * C] = st3
    o_ref[:, 2 * C:3 * C] = _relu(_dot(m2, wg_ref[1])) + h6_ref[...]
    o_ref[:, 3 * C:4 * C] = st3 + _relu(_dot(m2, wg_ref[2]))


def _tc_pre(S, W_pre, bn_gamma, bn_beta):
    h, stats = pl.pallas_call(
        _pre_kernel,
        grid=(2, NB),
        in_specs=[pl.BlockSpec((1, BR, C), lambda i, j: (i, j, 0)),
                  pl.BlockSpec((1, C, C), lambda i, j: (i, 0, 0))],
        out_specs=[pl.BlockSpec((1, BR, C), lambda i, j: (i, j, 0)),
                   pl.BlockSpec((1, 8, C), lambda i, j: (i, 0, 0))],
        out_shape=[jax.ShapeDtypeStruct((2, N, C), F32),
                   jax.ShapeDtypeStruct((2, 8, C), F32)],
    )(S, W_pre)
    g8 = jnp.broadcast_to(bn_gamma[:, None, :], (2, 8, C))
    b8 = jnp.broadcast_to(bn_beta[:, None, :], (2, 8, C))
    P = pl.pallas_call(
        _norm_kernel,
        grid=(2, NB),
        in_specs=[pl.BlockSpec((1, BR, C), lambda i, j: (i, j, 0)),
                  pl.BlockSpec((1, 8, C), lambda i, j: (i, 0, 0)),
                  pl.BlockSpec((1, 8, C), lambda i, j: (i, 0, 0)),
                  pl.BlockSpec((1, 8, C), lambda i, j: (i, 0, 0))],
        out_specs=pl.BlockSpec((1, BR, C), lambda i, j: (i, j, 0)),
        out_shape=jax.ShapeDtypeStruct((2, N, C), F32),
    )(h, stats, g8, b8)
    return P


def _tc_mid(P, a01, deg, W_sage, W_gcn):
    return pl.pallas_call(
        _mid_kernel,
        grid=(NB,),
        in_specs=[pl.BlockSpec((BR, C), lambda j: (j, 0)),
                  pl.BlockSpec((BR, C), lambda j: (j, 0)),
                  pl.BlockSpec((NSC, BR, C), lambda j: (0, j, 0)),
                  pl.BlockSpec((BR, 1), lambda j: (j, 0)),
                  pl.BlockSpec((3, 2, C, C), lambda j: (0, 0, 0, 0)),
                  pl.BlockSpec((3, C, C), lambda j: (0, 0, 0))],
        out_specs=[pl.BlockSpec((BR, C), lambda j: (j, 0))] * 3,
        out_shape=[jax.ShapeDtypeStruct((N, C), F32)] * 3,
    )(P[0], P[1], a01, deg, W_sage, W_gcn)


def _tc_fin(st2, st3, h6, a2, deg, W_gcn):
    return pl.pallas_call(
        _fin_kernel,
        grid=(NB,),
        in_specs=[pl.BlockSpec((BR, C), lambda j: (j, 0)),
                  pl.BlockSpec((BR, C), lambda j: (j, 0)),
                  pl.BlockSpec((BR, C), lambda j: (j, 0)),
                  pl.BlockSpec((NSC, BR, C), lambda j: (0, j, 0)),
                  pl.BlockSpec((BR, 1), lambda j: (j, 0)),
                  pl.BlockSpec((3, C, C), lambda j: (0, 0, 0))],
        out_specs=pl.BlockSpec((BR, 4 * C), lambda j: (j, 0)),
        out_shape=jax.ShapeDtypeStruct((N, 4 * C), F32),
    )(st2, st3, h6, a2, deg, W_gcn)


def kernel(s0, s1, edge_index, drop_prob, W_pre, bn_gamma, bn_beta,
           W_sage, W_gcn):
    src = edge_index[0].astype(jnp.int32)
    dst = edge_index[1].astype(jnp.int32)

    tot1 = NT * K1 * CHUNK
    src_p = jnp.concatenate(
        [src, jnp.zeros((tot1 - E,), jnp.int32)]).reshape(NT, K1, CHUNK)
    dst_p = jnp.concatenate(
        [dst, jnp.full((tot1 - E,), N, jnp.int32)]).reshape(NT, K1, CHUNK)
    slab1_src = jnp.concatenate([src_p, src_p + N]).reshape(NW, K1, CHUNK)
    slab1_dst = jnp.concatenate([dst_p, dst_p]).reshape(NW, K1, CHUNK)

    half = E // NSC
    pad2 = NT * K2 * CHUNK - half
    slab2_src = jnp.pad(src.reshape(NSC, half),
                        ((0, 0), (0, pad2))).reshape(NW, K2, CHUNK)
    slab2_dst = jnp.pad(dst.reshape(NSC, half), ((0, 0), (0, pad2)),
                        constant_values=N).reshape(NW, K2, CHUNK)

    zc = jnp.zeros((N_PAD, C), F32)

    P = _tc_pre(jnp.stack([s0, s1]), W_pre, bn_gamma, bn_beta)

    seg1 = _make_seg_kernel(K1, with_deg=True)
    a01, deg_flat = seg1(P.reshape(NSC * N, C), slab1_src, slab1_dst, zc)
    a01 = a01.reshape(NSC, N, C)
    deg = deg_flat.reshape(DN * C)[:N].reshape(N, 1)

    st2, st3, h6 = _tc_mid(P, a01, deg, W_sage, W_gcn)

    seg2 = _make_seg_kernel(K2, with_deg=False)
    (a2,) = seg2(st2, slab2_src, slab2_dst, zc)
    a2 = a2.reshape(NSC, N, C)

    return _tc_fin(st2, st3, h6, a2, deg, W_gcn)


# speedup vs baseline: 3.7447x; 1.1689x over previous
import jax
import jax.numpy as jnp
from jax import lax
from jax.experimental import pallas as pl
from jax.experimental.pallas import tpu as pltpu
from jax.experimental.pallas import tpu_sc as plsc

N = 10000
C = 128
E = 320000
F32 = jnp.float32

NSC = 2
NT = 16
NW = NSC * NT
CHUNK = 128
GRP = 8
K1 = 160
K2 = 80
N_PAD = 10112
ZSTRIPE = N_PAD // NT
OSTRIPE_LAST = N - (NT - 1) * ZSTRIPE
DN = 80

BR = 1000
NB = N // BR


def _make_seg_kernel(k_chunks, with_deg):
    mesh = plsc.VectorSubcoreMesh(core_axis_name="c", subcore_axis_name="s")
    out_type = [jax.ShapeDtypeStruct((NSC * N, C), F32)]
    scratch = [
        pltpu.VMEM((2, GRP, CHUNK), jnp.int32),
        pltpu.VMEM((2, GRP, CHUNK), jnp.int32),
        pltpu.VMEM((2, CHUNK, C), F32),
        pltpu.VMEM_SHARED((N_PAD, C), F32),
        pltpu.SemaphoreType.DMA,
        pltpu.SemaphoreType.DMA,
    ]
    if with_deg:
        out_type.append(jax.ShapeDtypeStruct((DN, C), F32))
        scratch += [
            pltpu.VMEM((DN, C), F32),
            pltpu.VMEM((DN,), jnp.int32),
            pltpu.VMEM_SHARED((DN, C), F32),
        ]

    def body(*refs):
        if with_deg:
            (table, srcs, dsts, zc, out, deg_out,
             src_v, dst_v, rows_v, acc_sh, sem_a, sem_b,
             deg_v, iden_v, deg_sh) = refs
        else:
            (table, srcs, dsts, zc, out,
             src_v, dst_v, rows_v, acc_sh, sem_a, sem_b) = refs
        sems = [sem_a, sem_b]
        c = lax.axis_index("c")
        s = lax.axis_index("s")
        w = c * NT + s
        zoff = pl.multiple_of(s * ZSTRIPE, 8)
        pltpu.sync_copy(zc.at[pl.ds(zoff, ZSTRIPE)],
                        acc_sh.at[pl.ds(zoff, ZSTRIPE)])
        if with_deg:
            @pl.when(s == 0)
            def _():
                pltpu.sync_copy(zc.at[pl.ds(0, DN)], deg_sh.at[...])
            zv = jnp.zeros((16,), F32)

            def zrow(i, carry):
                for k in range(C // 16):
                    deg_v[i, pl.ds(k * 16, 16)] = zv
                return carry

            lax.fori_loop(0, DN, zrow, 0)
            for k in range(DN // 16):
                iden_v[pl.ds(k * 16, 16)] = (
                    lax.iota(jnp.int32, 16) + (k * 16))
        plsc.subcore_barrier()

        ones16 = jnp.full((16,), 1.0, F32)
        npairs = k_chunks // (2 * GRP)

        def idx_load(ab, g):
            goff = pl.multiple_of(g * GRP, 8)
            pltpu.sync_copy(srcs.at[w, pl.ds(goff, GRP)], src_v.at[ab])
            pltpu.sync_copy(dsts.at[w, pl.ds(goff, GRP)], dst_v.at[ab])

        def fire(ab, q, par):
            pltpu.async_copy(table.at[src_v.at[ab, q]], rows_v.at[par],
                             sems[par])

        def consume(ab, q, par):
            pltpu.make_async_copy(table.at[src_v.at[0, 0]], rows_v.at[par],
                                  sems[par]).wait()
            pltpu.sync_copy(rows_v.at[par], acc_sh.at[dst_v.at[ab, q]],
                            add=True)
            if with_deg:
                for i in range(CHUNK // 16):
                    d16 = dst_v[ab, q, pl.ds(i * 16, 16)]
                    plsc.addupdate_scatter(
                        deg_v,
                        [lax.shift_right_logical(d16, 7),
                         lax.bitwise_and(d16, 127)],
                        ones16)

        idx_load(0, 0)
        fire(0, 0, 0)

        def pair(t, carry):
            idx_load(1, 2 * t + 1)
            for q in range(GRP):
                if q < GRP - 1:
                    fire(0, q + 1, (q + 1) % 2)
                else:
                    fire(1, 0, 0)
                consume(0, q, q % 2)

            @pl.when(t < npairs - 1)
            def _():
                idx_load(0, 2 * t + 2)

            for q in range(GRP):
                if q < GRP - 1:
                    fire(1, q + 1, (q + 1) % 2)
                else:
                    @pl.when(t < npairs - 1)
                    def _():
                        fire(0, 0, 0)
                consume(1, q, q % 2)
            return carry

        lax.fori_loop(0, npairs, pair, 0)
        if with_deg:
            pltpu.sync_copy(deg_v, deg_sh.at[iden_v], add=True)
        plsc.subcore_barrier()
        ooff = pl.multiple_of(c * N + s * ZSTRIPE, 8)

        @pl.when(s < NT - 1)
        def _():
            pltpu.sync_copy(acc_sh.at[pl.ds(zoff, ZSTRIPE)],
                            out.at[pl.ds(ooff, ZSTRIPE)])

        @pl.when(s == NT - 1)
        def _():
            pltpu.sync_copy(acc_sh.at[pl.ds((NT - 1) * ZSTRIPE, OSTRIPE_LAST)],
                            out.at[pl.ds(ooff, OSTRIPE_LAST)])

        if with_deg:
            @pl.when((c == 0) & (s == 0))
            def _():
                pltpu.sync_copy(deg_sh, deg_out)

    return pl.kernel(body, out_type=tuple(out_type), mesh=mesh,
                     scratch_types=scratch,
                     compiler_params=pltpu.CompilerParams(
                         needs_layout_passes=False))


def _dot(a, b):
    return jnp.dot(a, b, preferred_element_type=F32)


def _relu(x):
    return jnp.maximum(x, 0.0)


def _pre_kernel(s_ref, w_ref, h_ref, st_ref):
    j = pl.program_id(1)
    h = _dot(s_ref[0], w_ref[0])
    h_ref[0] = h
    colsum = jnp.sum(h, axis=0, keepdims=True)
    colsq = jnp.sum(h * h, axis=0, keepdims=True)
    stats = jnp.concatenate(
        [colsum, colsq, jnp.zeros((6, C), F32)], axis=0)

    @pl.when(j == 0)
    def _():
        st_ref[0] = stats

    @pl.when(j > 0)
    def _():
        st_ref[0] = st_ref[0] + stats


def _norm_kernel(h_ref, st_ref, g_ref, b_ref, p_ref):
    st = st_ref[0]
    mean = st[0:1] * (1.0 / N)
    var = st[1:2] * (1.0 / N) - mean * mean
    scale = g_ref[0, 0:1] * lax.rsqrt(var + 1e-5)
    shift = b_ref[0, 0:1] - mean * scale
    p_ref[0] = _relu(h_ref[0] * scale + shift)


def _mid_kernel(p0_ref, p1_ref, a_ref, deg_ref, ws_ref, wg_ref,
                st2_ref, st3_ref, h6_ref):
    p0 = p0_ref[...]
    p1 = p1_ref[...]
    r = 1.0 / (deg_ref[...] + 1.0)
    m0 = (a_ref[0] + p0) * r
    m1 = (a_ref[1] + p1) * r
    st2_ref[...] = (_relu(_dot(p0, ws_ref[0, 0]) + _dot(m0, ws_ref[0, 1]))
                    + _relu(_dot(m1, wg_ref[0])))
    st3_ref[...] = _relu(_dot(p1, ws_ref[1, 0]) + _dot(m1, ws_ref[1, 1])) + p0
    h6_ref[...] = _relu(_dot(p1, ws_ref[2, 0]) + _dot(m1, ws_ref[2, 1]))


def _fin_kernel(st2_ref, st3_ref, h6_ref, a2_ref, deg_ref, wg_ref, o_ref):
    st2 = st2_ref[...]
    st3 = st3_ref[...]
    r = 1.0 / (deg_ref[...] + 1.0)
    m2 = (a2_ref[0] + a2_ref[1] + st2) * r
    o_ref[:, 0:C] = st2
    o_ref[:, C:2 * C] = st3
    o_ref[:, 2 * C:3 * C] = _relu(_dot(m2, wg_ref[1])) + h6_ref[...]
    o_ref[:, 3 * C:4 * C] = st3 + _relu(_dot(m2, wg_ref[2]))


def _tc_pre(S, W_pre, bn_gamma, bn_beta):
    h, stats = pl.pallas_call(
        _pre_kernel,
        grid=(2, NB),
        in_specs=[pl.BlockSpec((1, BR, C), lambda i, j: (i, j, 0)),
                  pl.BlockSpec((1, C, C), lambda i, j: (i, 0, 0))],
        out_specs=[pl.BlockSpec((1, BR, C), lambda i, j: (i, j, 0)),
                   pl.BlockSpec((1, 8, C), lambda i, j: (i, 0, 0))],
        out_shape=[jax.ShapeDtypeStruct((2, N, C), F32),
                   jax.ShapeDtypeStruct((2, 8, C), F32)],
    )(S, W_pre)
    g8 = jnp.broadcast_to(bn_gamma[:, None, :], (2, 8, C))
    b8 = jnp.broadcast_to(bn_beta[:, None, :], (2, 8, C))
    P = pl.pallas_call(
        _norm_kernel,
        grid=(2, NB),
        in_specs=[pl.BlockSpec((1, BR, C), lambda i, j: (i, j, 0)),
                  pl.BlockSpec((1, 8, C), lambda i, j: (i, 0, 0)),
                  pl.BlockSpec((1, 8, C), lambda i, j: (i, 0, 0)),
                  pl.BlockSpec((1, 8, C), lambda i, j: (i, 0, 0))],
        out_specs=pl.BlockSpec((1, BR, C), lambda i, j: (i, j, 0)),
        out_shape=jax.ShapeDtypeStruct((2, N, C), F32),
    )(h, stats, g8, b8)
    return P


def _tc_mid(P, a01, deg, W_sage, W_gcn):
    return pl.pallas_call(
        _mid_kernel,
        grid=(NB,),
        in_specs=[pl.BlockSpec((BR, C), lambda j: (j, 0)),
                  pl.BlockSpec((BR, C), lambda j: (j, 0)),
                  pl.BlockSpec((NSC, BR, C), lambda j: (0, j, 0)),
                  pl.BlockSpec((BR, 1), lambda j: (j, 0)),
                  pl.BlockSpec((3, 2, C, C), lambda j: (0, 0, 0, 0)),
                  pl.BlockSpec((3, C, C), lambda j: (0, 0, 0))],
        out_specs=[pl.BlockSpec((BR, C), lambda j: (j, 0))] * 3,
        out_shape=[jax.ShapeDtypeStruct((N, C), F32)] * 3,
    )(P[0], P[1], a01, deg, W_sage, W_gcn)


def _tc_fin(st2, st3, h6, a2, deg, W_gcn):
    return pl.pallas_call(
        _fin_kernel,
        grid=(NB,),
        in_specs=[pl.BlockSpec((BR, C), lambda j: (j, 0)),
                  pl.BlockSpec((BR, C), lambda j: (j, 0)),
                  pl.BlockSpec((BR, C), lambda j: (j, 0)),
                  pl.BlockSpec((NSC, BR, C), lambda j: (0, j, 0)),
                  pl.BlockSpec((BR, 1), lambda j: (j, 0)),
                  pl.BlockSpec((3, C, C), lambda j: (0, 0, 0))],
        out_specs=pl.BlockSpec((BR, 4 * C), lambda j: (j, 0)),
        out_shape=jax.ShapeDtypeStruct((N, 4 * C), F32),
    )(st2, st3, h6, a2, deg, W_gcn)


def kernel(s0, s1, edge_index, drop_prob, W_pre, bn_gamma, bn_beta,
           W_sage, W_gcn):
    src = edge_index[0].astype(jnp.int32)
    dst = edge_index[1].astype(jnp.int32)

    tot1 = NT * K1 * CHUNK
    src_p = jnp.concatenate(
        [src, jnp.zeros((tot1 - E,), jnp.int32)]).reshape(NT, K1, CHUNK)
    dst_p = jnp.concatenate(
        [dst, jnp.full((tot1 - E,), N, jnp.int32)]).reshape(NT, K1, CHUNK)
    slab1_src = jnp.concatenate([src_p, src_p + N]).reshape(NW, K1, CHUNK)
    slab1_dst = jnp.concatenate([dst_p, dst_p]).reshape(NW, K1, CHUNK)

    half = E // NSC
    pad2 = NT * K2 * CHUNK - half
    slab2_src = jnp.pad(src.reshape(NSC, half),
                        ((0, 0), (0, pad2))).reshape(NW, K2, CHUNK)
    slab2_dst = jnp.pad(dst.reshape(NSC, half), ((0, 0), (0, pad2)),
                        constant_values=N).reshape(NW, K2, CHUNK)

    zc = jnp.zeros((N_PAD, C), F32)

    P = _tc_pre(jnp.stack([s0, s1]), W_pre, bn_gamma, bn_beta)

    seg1 = _make_seg_kernel(K1, with_deg=True)
    a01, deg_flat = seg1(P.reshape(NSC * N, C), slab1_src, slab1_dst, zc)
    a01 = a01.reshape(NSC, N, C)
    deg = deg_flat.reshape(DN * C)[:N].reshape(N, 1)

    st2, st3, h6 = _tc_mid(P, a01, deg, W_sage, W_gcn)

    seg2 = _make_seg_kernel(K2, with_deg=False)
    (a2,) = seg2(st2, slab2_src, slab2_dst, zc)
    a2 = a2.reshape(NSC, N, C)

    return _tc_fin(st2, st3, h6, a2, deg, W_gcn)


# speedup vs baseline: 3.7872x; 1.0113x over previous
import jax
import jax.numpy as jnp
from jax import lax
from jax.experimental import pallas as pl
from jax.experimental.pallas import tpu as pltpu
from jax.experimental.pallas import tpu_sc as plsc

N = 10000
C = 128
E = 320000
F32 = jnp.float32

NSC = 2
NT = 16
NW = NSC * NT
CHUNK = 128
GRP = 8
K1 = 160
K2 = 80
N_PAD = 10112
ZSTRIPE = N_PAD // NT
OSTRIPE_LAST = N - (NT - 1) * ZSTRIPE
DN = 80

BR = 1000
NB = N // BR


def _make_seg_kernel(k_chunks, with_deg):
    mesh = plsc.VectorSubcoreMesh(core_axis_name="c", subcore_axis_name="s")
    out_type = [jax.ShapeDtypeStruct((NSC * N, C), F32)]
    scratch = [
        pltpu.VMEM((2, GRP, CHUNK), jnp.int32),
        pltpu.VMEM((2, GRP, CHUNK), jnp.int32),
        pltpu.VMEM((2, CHUNK, C), F32),
        pltpu.VMEM_SHARED((N_PAD, C), F32),
        pltpu.SemaphoreType.DMA,
        pltpu.SemaphoreType.DMA,
        pltpu.SemaphoreType.DMA,
        pltpu.SemaphoreType.DMA,
    ]
    if with_deg:
        out_type.append(jax.ShapeDtypeStruct((NSC, DN, C), F32))
        scratch += [
            pltpu.VMEM((DN, C), F32),
            pltpu.VMEM((DN,), jnp.int32),
            pltpu.VMEM_SHARED((DN, C), F32),
        ]

    def body(*refs):
        if with_deg:
            (table, srcs, dsts, zc, out, deg_out,
             src_v, dst_v, rows_v, acc_sh, sem_a, sem_b, sem_c, sem_d,
             deg_v, iden_v, deg_sh) = refs
        else:
            (table, srcs, dsts, zc, out,
             src_v, dst_v, rows_v, acc_sh, sem_a, sem_b, sem_c, sem_d) = refs
        gsems = [sem_a, sem_b]
        ssems = [sem_c, sem_d]
        c = lax.axis_index("c")
        s = lax.axis_index("s")
        w = c * NT + s
        zoff = pl.multiple_of(s * ZSTRIPE, 8)
        pltpu.sync_copy(zc.at[pl.ds(zoff, ZSTRIPE)],
                        acc_sh.at[pl.ds(zoff, ZSTRIPE)])
        if with_deg:
            @pl.when(s == 0)
            def _():
                pltpu.sync_copy(zc.at[pl.ds(0, DN)], deg_sh.at[...])
            zv = jnp.zeros((16,), F32)

            def zrow(i, carry):
                for k in range(C // 16):
                    deg_v[i, pl.ds(k * 16, 16)] = zv
                return carry

            lax.fori_loop(0, DN, zrow, 0)
            for k in range(DN // 16):
                iden_v[pl.ds(k * 16, 16)] = (
                    lax.iota(jnp.int32, 16) + (k * 16))
        plsc.subcore_barrier()

        ones16 = jnp.full((16,), 1.0, F32)
        npairs = k_chunks // (2 * GRP)

        def idx_load(ab, g):
            goff = pl.multiple_of(g * GRP, 8)
            pltpu.sync_copy(srcs.at[w, pl.ds(goff, GRP)], src_v.at[ab])
            pltpu.sync_copy(dsts.at[w, pl.ds(goff, GRP)], dst_v.at[ab])

        def fire(ab, q, par):
            pltpu.async_copy(table.at[src_v.at[ab, q]], rows_v.at[par],
                             gsems[par])

        def chunk(ab, q, deg_pred, skip_wait1=False, fire_next=None,
                  sync_scatter=False):
            par = q % 2
            if not skip_wait1:
                pltpu.make_async_copy(
                    rows_v.at[1 - par], acc_sh.at[dst_v.at[0, 0]],
                    ssems[1 - par]).wait()
            if fire_next is not None:
                fire(fire_next[0], fire_next[1], 1 - par)
            pltpu.make_async_copy(table.at[src_v.at[0, 0]], rows_v.at[par],
                                  gsems[par]).wait()
            if sync_scatter:
                pltpu.sync_copy(rows_v.at[par], acc_sh.at[dst_v.at[ab, q]],
                                add=True)
            else:
                pltpu.async_copy(rows_v.at[par], acc_sh.at[dst_v.at[ab, q]],
                                 ssems[par], add=True)
            if with_deg:
                @pl.when(deg_pred)
                def _():
                    for i in range(CHUNK // 16):
                        d16 = dst_v[ab, q, pl.ds(i * 16, 16)]
                        plsc.addupdate_scatter(
                            deg_v,
                            [lax.shift_right_logical(d16, 7),
                             lax.bitwise_and(d16, 127)],
                            ones16)

        def pair_body(t, first, last):
            tb = jnp.asarray(t) < (npairs // 2)
            deg_pred = (((c == 0) & tb)
                        | ((c == 1) & jnp.logical_not(tb)))
            idx_load(1, 2 * t + 1)
            for q in range(GRP):
                chunk(0, q, deg_pred,
                      skip_wait1=(first and q == 0),
                      fire_next=(0, q + 1) if q < GRP - 1 else (1, 0))
            if not last:
                idx_load(0, 2 * t + 2)
            for q in range(GRP):
                if q < GRP - 1:
                    nxt = (1, q + 1)
                else:
                    nxt = None if last else (0, 0)
                chunk(1, q, deg_pred,
                      skip_wait1=(last and q == GRP - 1),
                      fire_next=nxt,
                      sync_scatter=(last and q >= GRP - 2))

        idx_load(0, 0)
        fire(0, 0, 0)
        pair_body(0, True, False)

        def pair(t, carry):
            pair_body(t, False, False)
            return carry

        lax.fori_loop(1, npairs - 1, pair, 0)
        pair_body(npairs - 1, False, True)
        if with_deg:
            pltpu.sync_copy(deg_v, deg_sh.at[iden_v], add=True)
        plsc.subcore_barrier()
        ooff = pl.multiple_of(c * N + s * ZSTRIPE, 8)

        @pl.when(s < NT - 1)
        def _():
            pltpu.sync_copy(acc_sh.at[pl.ds(zoff, ZSTRIPE)],
                            out.at[pl.ds(ooff, ZSTRIPE)])

        @pl.when(s == NT - 1)
        def _():
            pltpu.sync_copy(acc_sh.at[pl.ds((NT - 1) * ZSTRIPE, OSTRIPE_LAST)],
                            out.at[pl.ds(ooff, OSTRIPE_LAST)])

        if with_deg:
            @pl.when(s == 0)
            def _():
                pltpu.sync_copy(deg_sh, deg_out.at[c])

    return pl.kernel(body, out_type=tuple(out_type), mesh=mesh,
                     scratch_types=scratch,
                     compiler_params=pltpu.CompilerParams(
                         needs_layout_passes=False))


def _dot(a, b):
    return jnp.dot(a, b, preferred_element_type=F32)


def _relu(x):
    return jnp.maximum(x, 0.0)


def _pre_kernel(s_ref, w_ref, h_ref, st_ref):
    j = pl.program_id(1)
    h = _dot(s_ref[0], w_ref[0])
    h_ref[0] = h
    colsum = jnp.sum(h, axis=0, keepdims=True)
    colsq = jnp.sum(h * h, axis=0, keepdims=True)
    stats = jnp.concatenate(
        [colsum, colsq, jnp.zeros((6, C), F32)], axis=0)

    @pl.when(j == 0)
    def _():
        st_ref[0] = stats

    @pl.when(j > 0)
    def _():
        st_ref[0] = st_ref[0] + stats


def _norm_kernel(h_ref, st_ref, g_ref, b_ref, p_ref):
    st = st_ref[0]
    mean = st[0:1] * (1.0 / N)
    var = st[1:2] * (1.0 / N) - mean * mean
    scale = g_ref[0, 0:1] * lax.rsqrt(var + 1e-5)
    shift = b_ref[0, 0:1] - mean * scale
    p_ref[0] = _relu(h_ref[0] * scale + shift)


def _mid_kernel(p0_ref, p1_ref, a_ref, deg_ref, ws_ref, wg_ref,
                st2_ref, st3_ref, h6_ref):
    p0 = p0_ref[...]
    p1 = p1_ref[...]
    r = 1.0 / (deg_ref[0] + deg_ref[1] + 1.0)
    m0 = (a_ref[0] + p0) * r
    m1 = (a_ref[1] + p1) * r
    st2_ref[...] = (_relu(_dot(p0, ws_ref[0, 0]) + _dot(m0, ws_ref[0, 1]))
                    + _relu(_dot(m1, wg_ref[0])))
    st3_ref[...] = _relu(_dot(p1, ws_ref[1, 0]) + _dot(m1, ws_ref[1, 1])) + p0
    h6_ref[...] = _relu(_dot(p1, ws_ref[2, 0]) + _dot(m1, ws_ref[2, 1]))


def _fin_kernel(st2_ref, st3_ref, h6_ref, a2_ref, deg_ref, wg_ref, o_ref):
    st2 = st2_ref[...]
    st3 = st3_ref[...]
    r = 1.0 / (deg_ref[0] + deg_ref[1] + 1.0)
    m2 = (a2_ref[0] + a2_ref[1] + st2) * r
    o_ref[:, 0:C] = st2
    o_ref[:, C:2 * C] = st3
    o_ref[:, 2 * C:3 * C] = _relu(_dot(m2, wg_ref[1])) + h6_ref[...]
    o_ref[:, 3 * C:4 * C] = st3 + _relu(_dot(m2, wg_ref[2]))


def _tc_pre(S, W_pre, bn_gamma, bn_beta):
    h, stats = pl.pallas_call(
        _pre_kernel,
        grid=(2, NB),
        in_specs=[pl.BlockSpec((1, BR, C), lambda i, j: (i, j, 0)),
                  pl.BlockSpec((1, C, C), lambda i, j: (i, 0, 0))],
        out_specs=[pl.BlockSpec((1, BR, C), lambda i, j: (i, j, 0)),
                   pl.BlockSpec((1, 8, C), lambda i, j: (i, 0, 0))],
        out_shape=[jax.ShapeDtypeStruct((2, N, C), F32),
                   jax.ShapeDtypeStruct((2, 8, C), F32)],
    )(S, W_pre)
    g8 = jnp.broadcast_to(bn_gamma[:, None, :], (2, 8, C))
    b8 = jnp.broadcast_to(bn_beta[:, None, :], (2, 8, C))
    P = pl.pallas_call(
        _norm_kernel,
        grid=(2, NB),
        in_specs=[pl.BlockSpec((1, BR, C), lambda i, j: (i, j, 0)),
                  pl.BlockSpec((1, 8, C), lambda i, j: (i, 0, 0)),
                  pl.BlockSpec((1, 8, C), lambda i, j: (i, 0, 0)),
                  pl.BlockSpec((1, 8, C), lambda i, j: (i, 0, 0))],
        out_specs=pl.BlockSpec((1, BR, C), lambda i, j: (i, j, 0)),
        out_shape=jax.ShapeDtypeStruct((2, N, C), F32),
    )(h, stats, g8, b8)
    return P


def _tc_mid(P, a01, deg, W_sage, W_gcn):
    return pl.pallas_call(
        _mid_kernel,
        grid=(NB,),
        in_specs=[pl.BlockSpec((BR, C), lambda j: (j, 0)),
                  pl.BlockSpec((BR, C), lambda j: (j, 0)),
                  pl.BlockSpec((NSC, BR, C), lambda j: (0, j, 0)),
                  pl.BlockSpec((NSC, BR, 1), lambda j: (0, j, 0)),
                  pl.BlockSpec((3, 2, C, C), lambda j: (0, 0, 0, 0)),
                  pl.BlockSpec((3, C, C), lambda j: (0, 0, 0))],
        out_specs=[pl.BlockSpec((BR, C), lambda j: (j, 0))] * 3,
        out_shape=[jax.ShapeDtypeStruct((N, C), F32)] * 3,
    )(P[0], P[1], a01, deg, W_sage, W_gcn)


def _tc_fin(st2, st3, h6, a2, deg, W_gcn):
    return pl.pallas_call(
        _fin_kernel,
        grid=(NB,),
        in_specs=[pl.BlockSpec((BR, C), lambda j: (j, 0)),
                  pl.BlockSpec((BR, C), lambda j: (j, 0)),
                  pl.BlockSpec((BR, C), lambda j: (j, 0)),
                  pl.BlockSpec((NSC, BR, C), lambda j: (0, j, 0)),
                  pl.BlockSpec((NSC, BR, 1), lambda j: (0, j, 0)),
                  pl.BlockSpec((3, C, C), lambda j: (0, 0, 0))],
        out_specs=pl.BlockSpec((BR, 4 * C), lambda j: (j, 0)),
        out_shape=jax.ShapeDtypeStruct((N, 4 * C), F32),
    )(st2, st3, h6, a2, deg, W_gcn)


def kernel(s0, s1, edge_index, drop_prob, W_pre, bn_gamma, bn_beta,
           W_sage, W_gcn):
    src = edge_index[0].astype(jnp.int32)
    dst = edge_index[1].astype(jnp.int32)

    tot1 = NT * K1 * CHUNK
    src_p = jnp.concatenate(
        [src, jnp.zeros((tot1 - E,), jnp.int32)]).reshape(NT, K1, CHUNK)
    dst_p = jnp.concatenate(
        [dst, jnp.full((tot1 - E,), N, jnp.int32)]).reshape(NT, K1, CHUNK)
    slab1_src = jnp.concatenate([src_p, src_p + N]).reshape(NW, K1, CHUNK)
    slab1_dst = jnp.concatenate([dst_p, dst_p]).reshape(NW, K1, CHUNK)

    half = E // NSC
    pad2 = NT * K2 * CHUNK - half
    slab2_src = jnp.pad(src.reshape(NSC, half),
                        ((0, 0), (0, pad2))).reshape(NW, K2, CHUNK)
    slab2_dst = jnp.pad(dst.reshape(NSC, half), ((0, 0), (0, pad2)),
                        constant_values=N).reshape(NW, K2, CHUNK)

    zc = jnp.zeros((N_PAD, C), F32)

    P = _tc_pre(jnp.stack([s0, s1]), W_pre, bn_gamma, bn_beta)

    seg1 = _make_seg_kernel(K1, with_deg=True)
    a01, deg_flat = seg1(P.reshape(NSC * N, C), slab1_src, slab1_dst, zc)
    a01 = a01.reshape(NSC, N, C)
    deg = deg_flat.reshape(NSC, DN * C)[:, :N].reshape(NSC, N, 1)

    st2, st3, h6 = _tc_mid(P, a01, deg, W_sage, W_gcn)

    seg2 = _make_seg_kernel(K2, with_deg=False)
    (a2,) = seg2(st2, slab2_src, slab2_dst, zc)
    a2 = a2.reshape(NSC, N, C)

    return _tc_fin(st2, st3, h6, a2, deg, W_gcn)


# speedup vs baseline: 3.8048x; 1.0046x over previous
import jax
import jax.numpy as jnp
from jax import lax
from jax.experimental import pallas as pl
from jax.experimental.pallas import tpu as pltpu
from jax.experimental.pallas import tpu_sc as plsc

N = 10000
C = 128
E = 320000
F32 = jnp.float32

NSC = 2
NT = 16
NW = NSC * NT
CHUNK = 128
GRP = 8
K1 = 160
K2 = 80
N_PAD = 10112
ZSTRIPE = N_PAD // NT
OSTRIPE_LAST = N - (NT - 1) * ZSTRIPE
DN = 80

BR = 1000
NB = N // BR


def _make_seg_kernel(k_chunks, with_deg):
    mesh = plsc.VectorSubcoreMesh(core_axis_name="c", subcore_axis_name="s")
    out_type = [jax.ShapeDtypeStruct((NSC * N, C), F32)]
    scratch = [
        pltpu.VMEM((2, GRP, CHUNK), jnp.int32),
        pltpu.VMEM((2, GRP, CHUNK), jnp.int32),
        pltpu.VMEM((2, CHUNK, C), F32),
        pltpu.VMEM_SHARED((N_PAD, C), F32),
        pltpu.SemaphoreType.DMA,
        pltpu.SemaphoreType.DMA,
        pltpu.SemaphoreType.DMA,
        pltpu.SemaphoreType.DMA,
    ]
    if with_deg:
        out_type.append(jax.ShapeDtypeStruct((NSC, DN, C), F32))
        scratch += [
            pltpu.VMEM((DN, C), F32),
            pltpu.VMEM((DN,), jnp.int32),
            pltpu.VMEM_SHARED((DN, C), F32),
        ]

    def body(*refs):
        if with_deg:
            (table, srcs, dsts, zc, out, deg_out,
             src_v, dst_v, rows_v, acc_sh, sem_a, sem_b, sem_c, sem_d,
             deg_v, iden_v, deg_sh) = refs
        else:
            (table, srcs, dsts, zc, out,
             src_v, dst_v, rows_v, acc_sh, sem_a, sem_b, sem_c, sem_d) = refs
        gsems = [sem_a, sem_b]
        ssems = [sem_c, sem_d]
        c = lax.axis_index("c")
        s = lax.axis_index("s")
        w = c * NT + s
        zoff = pl.multiple_of(s * ZSTRIPE, 8)
        pltpu.sync_copy(zc.at[pl.ds(zoff, ZSTRIPE)],
                        acc_sh.at[pl.ds(zoff, ZSTRIPE)])
        if with_deg:
            @pl.when(s == 0)
            def _():
                pltpu.sync_copy(zc.at[pl.ds(0, DN)], deg_sh.at[...])
            zv = jnp.zeros((16,), F32)

            def zrow(i, carry):
                for k in range(C // 16):
                    deg_v[i, pl.ds(k * 16, 16)] = zv
                return carry

            lax.fori_loop(0, DN, zrow, 0)
            for k in range(DN // 16):
                iden_v[pl.ds(k * 16, 16)] = (
                    lax.iota(jnp.int32, 16) + (k * 16))
        plsc.subcore_barrier()

        ones16 = jnp.full((16,), 1.0, F32)
        npairs = k_chunks // (2 * GRP)

        def idx_load(ab, g):
            goff = pl.multiple_of(g * GRP, 8)
            pltpu.sync_copy(srcs.at[w, pl.ds(goff, GRP)], src_v.at[ab])
            pltpu.sync_copy(dsts.at[w, pl.ds(goff, GRP)], dst_v.at[ab])

        def fire(ab, q, par):
            pltpu.async_copy(table.at[src_v.at[ab, q]], rows_v.at[par],
                             gsems[par])

        def chunk(ab, q, deg_pred, skip_wait1=False, fire_next=None,
                  sync_scatter=False):
            par = q % 2
            if not skip_wait1:
                pltpu.make_async_copy(
                    rows_v.at[1 - par], acc_sh.at[dst_v.at[0, 0]],
                    ssems[1 - par]).wait()
            if fire_next is not None:
                fire(fire_next[0], fire_next[1], 1 - par)
            pltpu.make_async_copy(table.at[src_v.at[0, 0]], rows_v.at[par],
                                  gsems[par]).wait()
            if sync_scatter:
                pltpu.sync_copy(rows_v.at[par], acc_sh.at[dst_v.at[ab, q]],
                                add=True)
            else:
                pltpu.async_copy(rows_v.at[par], acc_sh.at[dst_v.at[ab, q]],
                                 ssems[par], add=True)
            if with_deg:
                @pl.when(deg_pred)
                def _():
                    for i in range(CHUNK // 16):
                        d16 = dst_v[ab, q, pl.ds(i * 16, 16)]
                        plsc.addupdate_scatter(
                            deg_v,
                            [lax.shift_right_logical(d16, 7),
                             lax.bitwise_and(d16, 127)],
                            ones16)

        def pair_body(t, first, last):
            tb = jnp.asarray(t) < (npairs // 2)
            deg_pred = (((c == 0) & tb)
                        | ((c == 1) & jnp.logical_not(tb)))
            idx_load(1, 2 * t + 1)
            for q in range(GRP):
                chunk(0, q, deg_pred,
                      skip_wait1=(first and q == 0),
                      fire_next=(0, q + 1) if q < GRP - 1 else (1, 0))
            if not last:
                idx_load(0, 2 * t + 2)
            for q in range(GRP):
                if q < GRP - 1:
                    nxt = (1, q + 1)
                else:
                    nxt = None if last else (0, 0)
                chunk(1, q, deg_pred,
                      skip_wait1=(last and q == GRP - 1),
                      fire_next=nxt,
                      sync_scatter=(last and q >= GRP - 2))

        idx_load(0, 0)
        fire(0, 0, 0)
        pair_body(0, True, False)

        def pair(t, carry):
            pair_body(t, False, False)
            return carry

        lax.fori_loop(1, npairs - 1, pair, 0)
        pair_body(npairs - 1, False, True)
        if with_deg:
            pltpu.sync_copy(deg_v, deg_sh.at[iden_v], add=True)
        plsc.subcore_barrier()
        ooff = pl.multiple_of(c * N + s * ZSTRIPE, 8)

        @pl.when(s < NT - 1)
        def _():
            pltpu.sync_copy(acc_sh.at[pl.ds(zoff, ZSTRIPE)],
                            out.at[pl.ds(ooff, ZSTRIPE)])

        @pl.when(s == NT - 1)
        def _():
            pltpu.sync_copy(acc_sh.at[pl.ds((NT - 1) * ZSTRIPE, OSTRIPE_LAST)],
                            out.at[pl.ds(ooff, OSTRIPE_LAST)])

        if with_deg:
            @pl.when(s == 0)
            def _():
                pltpu.sync_copy(deg_sh, deg_out.at[c])

    return pl.kernel(body, out_type=tuple(out_type), mesh=mesh,
                     scratch_types=scratch,
                     compiler_params=pltpu.CompilerParams(
                         needs_layout_passes=False))


def _dot(a, b):
    return jnp.dot(a, b, preferred_element_type=F32)


def _relu(x):
    return jnp.maximum(x, 0.0)


def _pre_kernel(s_ref, w_ref, h_ref, st_ref):
    j = pl.program_id(1)
    h = _dot(s_ref[0], w_ref[0])
    h_ref[0] = h
    colsum = jnp.sum(h, axis=0, keepdims=True)
    colsq = jnp.sum(h * h, axis=0, keepdims=True)
    stats = jnp.concatenate(
        [colsum, colsq, jnp.zeros((6, C), F32)], axis=0)

    @pl.when(j == 0)
    def _():
        st_ref[0] = stats

    @pl.when(j > 0)
    def _():
        st_ref[0] = st_ref[0] + stats


def _norm_kernel(h_ref, st_ref, g_ref, b_ref, p_ref):
    st = st_ref[0]
    mean = st[0:1] * (1.0 / N)
    var = st[1:2] * (1.0 / N) - mean * mean
    scale = g_ref[0, 0:1] * lax.rsqrt(var + 1e-5)
    shift = b_ref[0, 0:1] - mean * scale
    p_ref[0] = _relu(h_ref[0] * scale + shift)


def _mida_kernel(p0_ref, p1_ref, a_ref, deg_ref, ws_ref, wg_ref, st2_ref):
    p0 = p0_ref[...]
    p1 = p1_ref[...]
    r = 1.0 / (deg_ref[0] + deg_ref[1] + 1.0)
    m0 = (a_ref[0] + p0) * r
    m1 = (a_ref[1] + p1) * r
    st2_ref[...] = (_relu(_dot(p0, ws_ref[0, 0]) + _dot(m0, ws_ref[0, 1]))
                    + _relu(_dot(m1, wg_ref[0])))


def _midb_kernel(p0_ref, p1_ref, a_ref, deg_ref, ws_ref,
                 st3_ref, h6_ref):
    p0 = p0_ref[...]
    p1 = p1_ref[...]
    r = 1.0 / (deg_ref[0] + deg_ref[1] + 1.0)
    m1 = (a_ref[1] + p1) * r
    st3_ref[...] = _relu(_dot(p1, ws_ref[1, 0]) + _dot(m1, ws_ref[1, 1])) + p0
    h6_ref[...] = _relu(_dot(p1, ws_ref[2, 0]) + _dot(m1, ws_ref[2, 1]))


def _fin_kernel(st2_ref, st3_ref, h6_ref, a2_ref, deg_ref, wg_ref, o_ref):
    st2 = st2_ref[...]
    st3 = st3_ref[...]
    r = 1.0 / (deg_ref[0] + deg_ref[1] + 1.0)
    m2 = (a2_ref[0] + a2_ref[1] + st2) * r
    o_ref[:, 0:C] = st2
    o_ref[:, C:2 * C] = st3
    o_ref[:, 2 * C:3 * C] = _relu(_dot(m2, wg_ref[1])) + h6_ref[...]
    o_ref[:, 3 * C:4 * C] = st3 + _relu(_dot(m2, wg_ref[2]))


def _tc_pre(S, W_pre, bn_gamma, bn_beta):
    h, stats = pl.pallas_call(
        _pre_kernel,
        grid=(2, NB),
        in_specs=[pl.BlockSpec((1, BR, C), lambda i, j: (i, j, 0)),
                  pl.BlockSpec((1, C, C), lambda i, j: (i, 0, 0))],
        out_specs=[pl.BlockSpec((1, BR, C), lambda i, j: (i, j, 0)),
                   pl.BlockSpec((1, 8, C), lambda i, j: (i, 0, 0))],
        out_shape=[jax.ShapeDtypeStruct((2, N, C), F32),
                   jax.ShapeDtypeStruct((2, 8, C), F32)],
    )(S, W_pre)
    g8 = jnp.broadcast_to(bn_gamma[:, None, :], (2, 8, C))
    b8 = jnp.broadcast_to(bn_beta[:, None, :], (2, 8, C))
    P = pl.pallas_call(
        _norm_kernel,
        grid=(2, NB),
        in_specs=[pl.BlockSpec((1, BR, C), lambda i, j: (i, j, 0)),
                  pl.BlockSpec((1, 8, C), lambda i, j: (i, 0, 0)),
                  pl.BlockSpec((1, 8, C), lambda i, j: (i, 0, 0)),
                  pl.BlockSpec((1, 8, C), lambda i, j: (i, 0, 0))],
        out_specs=pl.BlockSpec((1, BR, C), lambda i, j: (i, j, 0)),
        out_shape=jax.ShapeDtypeStruct((2, N, C), F32),
    )(h, stats, g8, b8)
    return P


def _tc_mida(P, a01, deg, W_sage, W_gcn):
    return pl.pallas_call(
        _mida_kernel,
        grid=(NB,),
        in_specs=[pl.BlockSpec((BR, C), lambda j: (j, 0)),
                  pl.BlockSpec((BR, C), lambda j: (j, 0)),
                  pl.BlockSpec((NSC, BR, C), lambda j: (0, j, 0)),
                  pl.BlockSpec((NSC, BR, 1), lambda j: (0, j, 0)),
                  pl.BlockSpec((3, 2, C, C), lambda j: (0, 0, 0, 0)),
                  pl.BlockSpec((3, C, C), lambda j: (0, 0, 0))],
        out_specs=pl.BlockSpec((BR, C), lambda j: (j, 0)),
        out_shape=jax.ShapeDtypeStruct((N, C), F32),
    )(P[0], P[1], a01, deg, W_sage, W_gcn)


def _tc_midb(P, a01, deg, W_sage):
    return pl.pallas_call(
        _midb_kernel,
        grid=(NB,),
        in_specs=[pl.BlockSpec((BR, C), lambda j: (j, 0)),
                  pl.BlockSpec((BR, C), lambda j: (j, 0)),
                  pl.BlockSpec((NSC, BR, C), lambda j: (0, j, 0)),
                  pl.BlockSpec((NSC, BR, 1), lambda j: (0, j, 0)),
                  pl.BlockSpec((3, 2, C, C), lambda j: (0, 0, 0, 0))],
        out_specs=[pl.BlockSpec((BR, C), lambda j: (j, 0))] * 2,
        out_shape=[jax.ShapeDtypeStruct((N, C), F32)] * 2,
    )(P[0], P[1], a01, deg, W_sage)


def _tc_fin(st2, st3, h6, a2, deg, W_gcn):
    return pl.pallas_call(
        _fin_kernel,
        grid=(NB,),
        in_specs=[pl.BlockSpec((BR, C), lambda j: (j, 0)),
                  pl.BlockSpec((BR, C), lambda j: (j, 0)),
                  pl.BlockSpec((BR, C), lambda j: (j, 0)),
                  pl.BlockSpec((NSC, BR, C), lambda j: (0, j, 0)),
                  pl.BlockSpec((NSC, BR, 1), lambda j: (0, j, 0)),
                  pl.BlockSpec((3, C, C), lambda j: (0, 0, 0))],
        out_specs=pl.BlockSpec((BR, 4 * C), lambda j: (j, 0)),
        out_shape=jax.ShapeDtypeStruct((N, 4 * C), F32),
    )(st2, st3, h6, a2, deg, W_gcn)


def kernel(s0, s1, edge_index, drop_prob, W_pre, bn_gamma, bn_beta,
           W_sage, W_gcn):
    src = edge_index[0].astype(jnp.int32)
    dst = edge_index[1].astype(jnp.int32)

    tot1 = NT * K1 * CHUNK
    src_p = jnp.concatenate(
        [src, jnp.zeros((tot1 - E,), jnp.int32)]).reshape(NT, K1, CHUNK)
    dst_p = jnp.concatenate(
        [dst, jnp.full((tot1 - E,), N, jnp.int32)]).reshape(NT, K1, CHUNK)
    slab1_src = jnp.concatenate([src_p, src_p + N]).reshape(NW, K1, CHUNK)
    slab1_dst = jnp.concatenate([dst_p, dst_p]).reshape(NW, K1, CHUNK)

    half = E // NSC
    pad2 = NT * K2 * CHUNK - half
    slab2_src = jnp.pad(src.reshape(NSC, half),
                        ((0, 0), (0, pad2))).reshape(NW, K2, CHUNK)
    slab2_dst = jnp.pad(dst.reshape(NSC, half), ((0, 0), (0, pad2)),
                        constant_values=N).reshape(NW, K2, CHUNK)

    zc = jnp.zeros((N_PAD, C), F32)

    P = _tc_pre(jnp.stack([s0, s1]), W_pre, bn_gamma, bn_beta)

    seg1 = _make_seg_kernel(K1, with_deg=True)
    a01, deg_flat = seg1(P.reshape(NSC * N, C), slab1_src, slab1_dst, zc)
    a01 = a01.reshape(NSC, N, C)
    deg = deg_flat.reshape(NSC, DN * C)[:, :N].reshape(NSC, N, 1)

    st2 = _tc_mida(P, a01, deg, W_sage, W_gcn)

    seg2 = _make_seg_kernel(K2, with_deg=False)
    (a2,) = seg2(st2, slab2_src, slab2_dst, zc)
    st3, h6 = _tc_midb(P, a01, deg, W_sage)
    a2 = a2.reshape(NSC, N, C)

    return _tc_fin(st2, st3, h6, a2, deg, W_gcn)


# speedup vs baseline: 3.8214x; 1.0044x over previous
import jax
import jax.numpy as jnp
from jax import lax
from jax.experimental import pallas as pl
from jax.experimental.pallas import tpu as pltpu
from jax.experimental.pallas import tpu_sc as plsc

N = 10000
C = 128
E = 320000
F32 = jnp.float32

NSC = 2
NT = 16
NW = NSC * NT
CHUNK = 128
GRP = 8
K1 = 160
K2 = 80
N_PAD = 10112
ZSTRIPE = N_PAD // NT
OSTRIPE_LAST = N - (NT - 1) * ZSTRIPE
DN = 80

BR = 1000
NB = N // BR


def _make_seg_kernel(k_chunks, with_deg):
    mesh = plsc.VectorSubcoreMesh(core_axis_name="c", subcore_axis_name="s")
    out_type = [jax.ShapeDtypeStruct((NSC * N, C), F32)]
    scratch = [
        pltpu.VMEM((2, GRP, CHUNK), jnp.int32),
        pltpu.VMEM((2, GRP, CHUNK), jnp.int32),
        pltpu.VMEM((2, CHUNK, C), F32),
        pltpu.VMEM_SHARED((N_PAD, C), F32),
        pltpu.SemaphoreType.DMA,
        pltpu.SemaphoreType.DMA,
        pltpu.SemaphoreType.DMA,
        pltpu.SemaphoreType.DMA,
    ]
    if with_deg:
        out_type.append(jax.ShapeDtypeStruct((NSC, DN, C), F32))
        scratch += [
            pltpu.VMEM((DN, C), F32),
            pltpu.VMEM((DN,), jnp.int32),
            pltpu.VMEM_SHARED((DN, C), F32),
        ]

    def body(*refs):
        if with_deg:
            (table, srcs, dsts, zc, out, deg_out,
             src_v, dst_v, rows_v, acc_sh, sem_a, sem_b, sem_c, sem_d,
             deg_v, iden_v, deg_sh) = refs
        else:
            (table, srcs, dsts, zc, out,
             src_v, dst_v, rows_v, acc_sh, sem_a, sem_b, sem_c, sem_d) = refs
        gsems = [sem_a, sem_b]
        ssems = [sem_c, sem_d]
        c = lax.axis_index("c")
        s = lax.axis_index("s")
        w = c * NT + s
        zoff = pl.multiple_of(s * ZSTRIPE, 8)
        pltpu.sync_copy(zc.at[pl.ds(zoff, ZSTRIPE)],
                        acc_sh.at[pl.ds(zoff, ZSTRIPE)])
        if with_deg:
            @pl.when(s == 0)
            def _():
                pltpu.sync_copy(zc.at[pl.ds(0, DN)], deg_sh.at[...])
            zv = jnp.zeros((16,), F32)

            def zrow(i, carry):
                for k in range(C // 16):
                    deg_v[i, pl.ds(k * 16, 16)] = zv
                return carry

            lax.fori_loop(0, DN, zrow, 0)
            for k in range(DN // 16):
                iden_v[pl.ds(k * 16, 16)] = (
                    lax.iota(jnp.int32, 16) + (k * 16))
        plsc.subcore_barrier()

        ones16 = jnp.full((16,), 1.0, F32)
        npairs = k_chunks // (2 * GRP)

        def idx_load(ab, g):
            goff = pl.multiple_of(g * GRP, 8)
            pltpu.sync_copy(srcs.at[w, pl.ds(goff, GRP)], src_v.at[ab])
            pltpu.sync_copy(dsts.at[w, pl.ds(goff, GRP)], dst_v.at[ab])

        def fire(ab, q, par):
            pltpu.async_copy(table.at[src_v.at[ab, q]], rows_v.at[par],
                             gsems[par])

        def chunk(ab, q, deg_pred, skip_wait1=False, fire_next=None,
                  sync_scatter=False):
            par = q % 2
            if not skip_wait1:
                pltpu.make_async_copy(
                    rows_v.at[1 - par], acc_sh.at[dst_v.at[0, 0]],
                    ssems[1 - par]).wait()
            if fire_next is not None:
                fire(fire_next[0], fire_next[1], 1 - par)
            pltpu.make_async_copy(table.at[src_v.at[0, 0]], rows_v.at[par],
                                  gsems[par]).wait()
            if sync_scatter:
                pltpu.sync_copy(rows_v.at[par], acc_sh.at[dst_v.at[ab, q]],
                                add=True)
            else:
                pltpu.async_copy(rows_v.at[par], acc_sh.at[dst_v.at[ab, q]],
                                 ssems[par], add=True)
            if with_deg:
                @pl.when(deg_pred)
                def _():
                    for i in range(CHUNK // 16):
                        d16 = dst_v[ab, q, pl.ds(i * 16, 16)]
                        plsc.addupdate_scatter(
                            deg_v,
                            [lax.shift_right_logical(d16, 7),
                             lax.bitwise_and(d16, 127)],
                            ones16)

        def pair_body(t, first, last):
            tb = jnp.asarray(t) < (npairs // 2)
            deg_pred = (((c == 1) & tb)
                        | ((c == 0) & jnp.logical_not(tb)))
            idx_load(1, 2 * t + 1)
            for q in range(GRP):
                chunk(0, q, deg_pred,
                      skip_wait1=(first and q == 0),
                      fire_next=(0, q + 1) if q < GRP - 1 else (1, 0))
            if not last:
                idx_load(0, 2 * t + 2)
            for q in range(GRP):
                if q < GRP - 1:
                    nxt = (1, q + 1)
                else:
                    nxt = None if last else (0, 0)
                chunk(1, q, deg_pred,
                      skip_wait1=(last and q == GRP - 1),
                      fire_next=nxt,
                      sync_scatter=(last and q >= GRP - 2))

        idx_load(0, 0)
        fire(0, 0, 0)
        pair_body(0, True, False)

        def pair(t, carry):
            pair_body(t, False, False)
            return carry

        lax.fori_loop(1, npairs - 1, pair, 0)
        pair_body(npairs - 1, False, True)
        if with_deg:
            pltpu.sync_copy(deg_v, deg_sh.at[iden_v], add=True)
        plsc.subcore_barrier()
        ooff = pl.multiple_of(c * N + s * ZSTRIPE, 8)

        @pl.when(s < NT - 1)
        def _():
            pltpu.sync_copy(acc_sh.at[pl.ds(zoff, ZSTRIPE)],
                            out.at[pl.ds(ooff, ZSTRIPE)])

        @pl.when(s == NT - 1)
        def _():
            pltpu.sync_copy(acc_sh.at[pl.ds((NT - 1) * ZSTRIPE, OSTRIPE_LAST)],
                            out.at[pl.ds(ooff, OSTRIPE_LAST)])

        if with_deg:
            @pl.when(s == 0)
            def _():
                pltpu.sync_copy(deg_sh, deg_out.at[c])

    return pl.kernel(body, out_type=tuple(out_type), mesh=mesh,
                     scratch_types=scratch,
                     compiler_params=pltpu.CompilerParams(
                         needs_layout_passes=False))


def _dot(a, b):
    return jnp.dot(a, b, preferred_element_type=F32)


def _relu(x):
    return jnp.maximum(x, 0.0)


def _pre_kernel(s_ref, w_ref, h_ref, st_ref):
    j = pl.program_id(1)
    h = _dot(s_ref[0], w_ref[0])
    h_ref[0] = h
    colsum = jnp.sum(h, axis=0, keepdims=True)
    colsq = jnp.sum(h * h, axis=0, keepdims=True)
    stats = jnp.concatenate(
        [colsum, colsq, jnp.zeros((6, C), F32)], axis=0)

    @pl.when(j == 0)
    def _():
        st_ref[0] = stats

    @pl.when(j > 0)
    def _():
        st_ref[0] = st_ref[0] + stats


def _norm_kernel(h_ref, st_ref, g_ref, b_ref, p_ref):
    st = st_ref[0]
    mean = st[0:1] * (1.0 / N)
    var = st[1:2] * (1.0 / N) - mean * mean
    scale = g_ref[0, 0:1] * lax.rsqrt(var + 1e-5)
    shift = b_ref[0, 0:1] - mean * scale
    p_ref[0] = _relu(h_ref[0] * scale + shift)


def _mida_kernel(p0_ref, p1_ref, a_ref, deg_ref, ws_ref, wg_ref, st2_ref):
    p0 = p0_ref[...]
    p1 = p1_ref[...]
    r = 1.0 / (deg_ref[0] + deg_ref[1] + 1.0)
    m0 = (a_ref[0] + p0) * r
    m1 = (a_ref[1] + p1) * r
    st2_ref[...] = (_relu(_dot(p0, ws_ref[0, 0]) + _dot(m0, ws_ref[0, 1]))
                    + _relu(_dot(m1, wg_ref[0])))


def _midb_kernel(p0_ref, p1_ref, a_ref, deg_ref, ws_ref,
                 st3_ref, h6_ref):
    p0 = p0_ref[...]
    p1 = p1_ref[...]
    r = 1.0 / (deg_ref[0] + deg_ref[1] + 1.0)
    m1 = (a_ref[1] + p1) * r
    st3_ref[...] = _relu(_dot(p1, ws_ref[1, 0]) + _dot(m1, ws_ref[1, 1])) + p0
    h6_ref[...] = _relu(_dot(p1, ws_ref[2, 0]) + _dot(m1, ws_ref[2, 1]))


def _fin_kernel(st2_ref, st3_ref, h6_ref, a2_ref, deg_ref, wg_ref, o_ref):
    st2 = st2_ref[...]
    st3 = st3_ref[...]
    r = 1.0 / (deg_ref[0] + deg_ref[1] + 1.0)
    m2 = (a2_ref[0] + a2_ref[1] + st2) * r
    o_ref[:, 0:C] = st2
    o_ref[:, C:2 * C] = st3
    o_ref[:, 2 * C:3 * C] = _relu(_dot(m2, wg_ref[1])) + h6_ref[...]
    o_ref[:, 3 * C:4 * C] = st3 + _relu(_dot(m2, wg_ref[2]))


def _tc_pre(S, W_pre, bn_gamma, bn_beta):
    h, stats = pl.pallas_call(
        _pre_kernel,
        grid=(2, NB),
        in_specs=[pl.BlockSpec((1, BR, C), lambda i, j: (i, j, 0)),
                  pl.BlockSpec((1, C, C), lambda i, j: (i, 0, 0))],
        out_specs=[pl.BlockSpec((1, BR, C), lambda i, j: (i, j, 0)),
                   pl.BlockSpec((1, 8, C), lambda i, j: (i, 0, 0))],
        out_shape=[jax.ShapeDtypeStruct((2, N, C), F32),
                   jax.ShapeDtypeStruct((2, 8, C), F32)],
    )(S, W_pre)
    g8 = jnp.broadcast_to(bn_gamma[:, None, :], (2, 8, C))
    b8 = jnp.broadcast_to(bn_beta[:, None, :], (2, 8, C))
    P = pl.pallas_call(
        _norm_kernel,
        grid=(2, NB),
        in_specs=[pl.BlockSpec((1, BR, C), lambda i, j: (i, j, 0)),
                  pl.BlockSpec((1, 8, C), lambda i, j: (i, 0, 0)),
                  pl.BlockSpec((1, 8, C), lambda i, j: (i, 0, 0)),
                  pl.BlockSpec((1, 8, C), lambda i, j: (i, 0, 0))],
        out_specs=pl.BlockSpec((1, BR, C), lambda i, j: (i, j, 0)),
        out_shape=jax.ShapeDtypeStruct((2, N, C), F32),
    )(h, stats, g8, b8)
    return P


def _tc_mida(P, a01, deg, W_sage, W_gcn):
    return pl.pallas_call(
        _mida_kernel,
        grid=(NB,),
        in_specs=[pl.BlockSpec((BR, C), lambda j: (j, 0)),
                  pl.BlockSpec((BR, C), lambda j: (j, 0)),
                  pl.BlockSpec((NSC, BR, C), lambda j: (0, j, 0)),
                  pl.BlockSpec((NSC, BR, 1), lambda j: (0, j, 0)),
                  pl.BlockSpec((3, 2, C, C), lambda j: (0, 0, 0, 0)),
                  pl.BlockSpec((3, C, C), lambda j: (0, 0, 0))],
        out_specs=pl.BlockSpec((BR, C), lambda j: (j, 0)),
        out_shape=jax.ShapeDtypeStruct((N, C), F32),
    )(P[0], P[1], a01, deg, W_sage, W_gcn)


def _tc_midb(P, a01, deg, W_sage):
    return pl.pallas_call(
        _midb_kernel,
        grid=(NB,),
        in_specs=[pl.BlockSpec((BR, C), lambda j: (j, 0)),
                  pl.BlockSpec((BR, C), lambda j: (j, 0)),
                  pl.BlockSpec((NSC, BR, C), lambda j: (0, j, 0)),
                  pl.BlockSpec((NSC, BR, 1), lambda j: (0, j, 0)),
                  pl.BlockSpec((3, 2, C, C), lambda j: (0, 0, 0, 0))],
        out_specs=[pl.BlockSpec((BR, C), lambda j: (j, 0))] * 2,
        out_shape=[jax.ShapeDtypeStruct((N, C), F32)] * 2,
    )(P[0], P[1], a01, deg, W_sage)


def _tc_fin(st2, st3, h6, a2, deg, W_gcn):
    return pl.pallas_call(
        _fin_kernel,
        grid=(NB,),
        in_specs=[pl.BlockSpec((BR, C), lambda j: (j, 0)),
                  pl.BlockSpec((BR, C), lambda j: (j, 0)),
                  pl.BlockSpec((BR, C), lambda j: (j, 0)),
                  pl.BlockSpec((NSC, BR, C), lambda j: (0, j, 0)),
                  pl.BlockSpec((NSC, BR, 1), lambda j: (0, j, 0)),
                  pl.BlockSpec((3, C, C), lambda j: (0, 0, 0))],
        out_specs=pl.BlockSpec((BR, 4 * C), lambda j: (j, 0)),
        out_shape=jax.ShapeDtypeStruct((N, 4 * C), F32),
    )(st2, st3, h6, a2, deg, W_gcn)


def kernel(s0, s1, edge_index, drop_prob, W_pre, bn_gamma, bn_beta,
           W_sage, W_gcn):
    src = edge_index[0].astype(jnp.int32)
    dst = edge_index[1].astype(jnp.int32)

    tot1 = NT * K1 * CHUNK
    src_p = jnp.concatenate(
        [src, jnp.zeros((tot1 - E,), jnp.int32)]).reshape(NT, K1, CHUNK)
    dst_p = jnp.concatenate(
        [dst, jnp.full((tot1 - E,), N, jnp.int32)]).reshape(NT, K1, CHUNK)
    slab1_src = jnp.concatenate([src_p, src_p + N]).reshape(NW, K1, CHUNK)
    slab1_dst = jnp.concatenate([dst_p, dst_p]).reshape(NW, K1, CHUNK)

    half = E // NSC
    pad2 = NT * K2 * CHUNK - half
    slab2_src = jnp.pad(src.reshape(NSC, half),
                        ((0, 0), (0, pad2))).reshape(NW, K2, CHUNK)
    slab2_dst = jnp.pad(dst.reshape(NSC, half), ((0, 0), (0, pad2)),
                        constant_values=N).reshape(NW, K2, CHUNK)

    zc = jnp.zeros((N_PAD, C), F32)

    P = _tc_pre(jnp.stack([s0, s1]), W_pre, bn_gamma, bn_beta)

    seg1 = _make_seg_kernel(K1, with_deg=True)
    a01, deg_flat = seg1(P.reshape(NSC * N, C), slab1_src, slab1_dst, zc)
    a01 = a01.reshape(NSC, N, C)
    deg = deg_flat.reshape(NSC, DN * C)[:, :N].reshape(NSC, N, 1)

    st2 = _tc_mida(P, a01, deg, W_sage, W_gcn)

    seg2 = _make_seg_kernel(K2, with_deg=False)
    (a2,) = seg2(st2, slab2_src, slab2_dst, zc)
    st3, h6 = _tc_midb(P, a01, deg, W_sage)
    a2 = a2.reshape(NSC, N, C)

    return _tc_fin(st2, st3, h6, a2, deg, W_gcn)
